# baseline (speedup 1.0000x reference)
import jax
import jax.numpy as jnp
from jax import lax
from jax.experimental import pallas as pl
from jax.experimental.pallas import tpu as pltpu

_B, _L, _D = 4, 1024, 256
_C = 256
_NC = _L // _C

_PREC = None


def _dot_tt(a, b):
    return lax.dot_general(a, b, (((1,), (1,)), ((), ())),
                           preferred_element_type=jnp.float32,
                           precision=_PREC)


def _body(q_ref, k_ref, pprev_ref, gain_ref, oscale_ref, tri_ref, w2_ref,
          qout_ref, pfin_ref, p_scr, f_scr):
    c = pl.program_id(1)

    @pl.when(c == 0)
    def _():
        p0 = pprev_ref[0]
        p_scr[...] = p0
        f_scr[0, 0] = jnp.sum(p0 * p0)

    P = p_scr[...]
    F_in = f_scr[0, 0]
    Q = q_ref[0]
    K = k_ref[0]
    QK = jnp.concatenate([Q, K], axis=0)
    tri = tri_ref[...]
    W2 = w2_ref[...]

    A = _dot_tt(QK, P)
    num_inter = A[:_C]
    d = jnp.sum(A[_C:] * K, axis=1, keepdims=True)

    T = _dot_tt(QK, K)
    S_causal = T[:_C] * tri
    G = T[_C:]
    num_intra = jnp.dot(S_causal, K,
                        preferred_element_type=jnp.float32,
                        precision=_PREC)
    numer = num_inter + num_intra

    w_row = jnp.sum(G * G * W2, axis=1, keepdims=True)
    v = 2.0 * d + w_row
    cs = jnp.dot(tri, v, preferred_element_type=jnp.float32,
                 precision=_PREC)
    fro = jnp.sqrt(F_in + cs)

    q_aligned = numer * (1.0 / (fro + 1e-7))
    gain = jnp.exp(gain_ref[...])
    qout_ref[0] = jnp.tanh(q_aligned * gain) * oscale_ref[...]

    P_new = P + lax.dot_general(K, K, (((0,), (0,)), ((), ())),
                                preferred_element_type=jnp.float32,
                                precision=_PREC)
    p_scr[...] = P_new
    f_scr[0, 0] = F_in + jnp.sum(v)

    @pl.when(c == _NC - 1)
    def _():
        pfin_ref[0] = P_new


@jax.jit
def kernel(q, k, P_prev, log_gain, output_scale):
    gain2d = log_gain.reshape(1, _D)
    oscale2d = output_scale.reshape(1, _D)

    row = lax.broadcasted_iota(jnp.int32, (_C, _C), 0)
    col = lax.broadcasted_iota(jnp.int32, (_C, _C), 1)
    tri = (col <= row).astype(jnp.float32)
    w2 = tri + (col < row).astype(jnp.float32)

    q_out, P_final = pl.pallas_call(
        _body,
        out_shape=(
            jax.ShapeDtypeStruct((_B, _L, _D), jnp.float32),
            jax.ShapeDtypeStruct((_B, _D, _D), jnp.float32),
        ),
        grid=(_B, _NC),
        in_specs=[
            pl.BlockSpec((1, _C, _D), lambda b, c: (b, c, 0)),
            pl.BlockSpec((1, _C, _D), lambda b, c: (b, c, 0)),
            pl.BlockSpec((1, _D, _D), lambda b, c: (b, 0, 0)),
            pl.BlockSpec((1, _D), lambda b, c: (0, 0)),
            pl.BlockSpec((1, _D), lambda b, c: (0, 0)),
            pl.BlockSpec((_C, _C), lambda b, c: (0, 0)),
            pl.BlockSpec((_C, _C), lambda b, c: (0, 0)),
        ],
        out_specs=(
            pl.BlockSpec((1, _C, _D), lambda b, c: (b, c, 0)),
            pl.BlockSpec((1, _D, _D), lambda b, c: (b, 0, 0)),
        ),
        scratch_shapes=[
            pltpu.VMEM((_D, _D), jnp.float32),
            pltpu.SMEM((1, 1), jnp.float32),
        ],
        compiler_params=pltpu.CompilerParams(
            dimension_semantics=("parallel", "arbitrary"),
        ),
        name="qkproj_chunked",
    )(q, k, P_prev, gain2d, oscale2d, tri, w2)
    return q_out, P_final


# speedup vs baseline: 1.0158x; 1.0158x over previous
import jax
import jax.numpy as jnp
from jax import lax
from jax.experimental import pallas as pl
from jax.experimental.pallas import tpu as pltpu

_B, _L, _D = 4, 1024, 256
_C = 256
_NC = _L // _C

_PREC = None


def _dot_tt(a, b):
    return lax.dot_general(a, b, (((1,), (1,)), ((), ())),
                           preferred_element_type=jnp.float32,
                           precision=_PREC)


def _body(q_ref, k_ref, pprev_ref, gain_ref, oscale_ref, tri_ref, w2_ref,
          qout_ref, pfin_ref, p_scr):
    c = pl.program_id(1)

    @pl.when(c == 0)
    def _():
        p_scr[...] = pprev_ref[0]

    P = p_scr[...]
    F_in = jnp.sum(P * P)
    Q = q_ref[0]
    K = k_ref[0]
    QK = jnp.concatenate([Q, K], axis=0)
    tri = tri_ref[...]
    W2 = w2_ref[...]

    A = _dot_tt(QK, P)
    num_inter = A[:_C]
    d = jnp.sum(A[_C:] * K, axis=1, keepdims=True)

    T = _dot_tt(QK, K)
    S_causal = T[:_C] * tri
    G = T[_C:]
    num_intra = jnp.dot(S_causal, K,
                        preferred_element_type=jnp.float32,
                        precision=_PREC)
    numer = num_inter + num_intra

    w_row = jnp.sum(G * G * W2, axis=1, keepdims=True)
    v = 2.0 * d + w_row
    cs = jnp.dot(tri, v, preferred_element_type=jnp.float32,
                 precision=_PREC)
    fro = jnp.sqrt(F_in + cs)

    q_aligned = numer * (1.0 / (fro + 1e-7))
    gain = jnp.exp(gain_ref[...])
    qout_ref[0] = jnp.tanh(q_aligned * gain) * oscale_ref[...]

    P_new = P + lax.dot_general(K, K, (((0,), (0,)), ((), ())),
                                preferred_element_type=jnp.float32,
                                precision=_PREC)
    p_scr[...] = P_new

    @pl.when(c == _NC - 1)
    def _():
        pfin_ref[0] = P_new


@jax.jit
def kernel(q, k, P_prev, log_gain, output_scale):
    gain2d = log_gain.reshape(1, _D)
    oscale2d = output_scale.reshape(1, _D)

    row = lax.broadcasted_iota(jnp.int32, (_C, _C), 0)
    col = lax.broadcasted_iota(jnp.int32, (_C, _C), 1)
    tri = (col <= row).astype(jnp.float32)
    w2 = tri + (col < row).astype(jnp.float32)

    q_out, P_final = pl.pallas_call(
        _body,
        out_shape=(
            jax.ShapeDtypeStruct((_B, _L, _D), jnp.float32),
            jax.ShapeDtypeStruct((_B, _D, _D), jnp.float32),
        ),
        grid=(_B, _NC),
        in_specs=[
            pl.BlockSpec((1, _C, _D), lambda b, c: (b, c, 0)),
            pl.BlockSpec((1, _C, _D), lambda b, c: (b, c, 0)),
            pl.BlockSpec((1, _D, _D), lambda b, c: (b, 0, 0)),
            pl.BlockSpec((1, _D), lambda b, c: (0, 0)),
            pl.BlockSpec((1, _D), lambda b, c: (0, 0)),
            pl.BlockSpec((_C, _C), lambda b, c: (0, 0)),
            pl.BlockSpec((_C, _C), lambda b, c: (0, 0)),
        ],
        out_specs=(
            pl.BlockSpec((1, _C, _D), lambda b, c: (b, c, 0)),
            pl.BlockSpec((1, _D, _D), lambda b, c: (b, 0, 0)),
        ),
        scratch_shapes=[
            pltpu.VMEM((_D, _D), jnp.float32),
        ],
        compiler_params=pltpu.CompilerParams(
            dimension_semantics=("parallel", "arbitrary"),
        ),
        name="qkproj_chunked",
    )(q, k, P_prev, gain2d, oscale2d, tri, w2)
    return q_out, P_final


# speedup vs baseline: 1.1044x; 1.0873x over previous
import jax
import jax.numpy as jnp
from jax import lax
from jax.experimental import pallas as pl
from jax.experimental.pallas import tpu as pltpu

_B, _L, _D = 4, 1024, 256
_C = 256
_NC = _L // _C

_PREC = None


def _dot_tt(a, b):
    return lax.dot_general(a, b, (((1,), (1,)), ((), ())),
                           preferred_element_type=jnp.float32,
                           precision=_PREC)


def _body(q_ref, k_ref, pprev_ref, gain_ref, oscale_ref,
          qout_ref, pfin_ref, p_scr):
    c = pl.program_id(1)

    @pl.when(c == 0)
    def _():
        p_scr[...] = pprev_ref[0]

    P = p_scr[...]
    F_in = jnp.sum(P * P)
    Q = q_ref[0]
    K = k_ref[0]
    QK = jnp.concatenate([Q, K], axis=0)

    row = lax.broadcasted_iota(jnp.int32, (_C, _C), 0)
    col = lax.broadcasted_iota(jnp.int32, (_C, _C), 1)

    A = _dot_tt(QK, P)
    num_inter = A[:_C]
    d = jnp.sum(A[_C:] * K, axis=1, keepdims=True)

    T = _dot_tt(QK, K)
    S = T[:_C]
    G = T[_C:]
    S_causal = jnp.where(col <= row, S, 0.0)
    num_intra = jnp.dot(S_causal, K,
                        preferred_element_type=jnp.float32,
                        precision=_PREC)
    numer = num_inter + num_intra

    G2 = G * G
    W = jnp.where(col < row, 2.0, jnp.where(col == row, 1.0, 0.0))
    w_row = jnp.sum(G2 * W, axis=1, keepdims=True)

    tri = jnp.where(col <= row, 1.0, 0.0)
    cs = jnp.dot(tri, 2.0 * d + w_row,
                 preferred_element_type=jnp.float32,
                 precision=_PREC)
    fro = jnp.sqrt(F_in + cs)

    q_aligned = numer * (1.0 / (fro + 1e-7))
    gain = jnp.exp(gain_ref[...])
    qout_ref[0] = jnp.tanh(q_aligned * gain) * oscale_ref[...]

    P_new = P + lax.dot_general(K, K, (((0,), (0,)), ((), ())),
                                preferred_element_type=jnp.float32,
                                precision=_PREC)
    p_scr[...] = P_new

    @pl.when(c == _NC - 1)
    def _():
        pfin_ref[0] = P_new


@jax.jit
def kernel(q, k, P_prev, log_gain, output_scale):
    gain2d = log_gain.reshape(1, _D)
    oscale2d = output_scale.reshape(1, _D)

    q_out, P_final = pl.pallas_call(
        _body,
        out_shape=(
            jax.ShapeDtypeStruct((_B, _L, _D), jnp.float32),
            jax.ShapeDtypeStruct((_B, _D, _D), jnp.float32),
        ),
        grid=(_B, _NC),
        in_specs=[
            pl.BlockSpec((1, _C, _D), lambda b, c: (b, c, 0)),
            pl.BlockSpec((1, _C, _D), lambda b, c: (b, c, 0)),
            pl.BlockSpec((1, _D, _D), lambda b, c: (b, 0, 0)),
            pl.BlockSpec((1, _D), lambda b, c: (0, 0)),
            pl.BlockSpec((1, _D), lambda b, c: (0, 0)),
        ],
        out_specs=(
            pl.BlockSpec((1, _C, _D), lambda b, c: (b, c, 0)),
            pl.BlockSpec((1, _D, _D), lambda b, c: (b, 0, 0)),
        ),
        scratch_shapes=[
            pltpu.VMEM((_D, _D), jnp.float32),
        ],
        compiler_params=pltpu.CompilerParams(
            dimension_semantics=("parallel", "arbitrary"),
        ),
        name="qkproj_chunked",
    )(q, k, P_prev, gain2d, oscale2d)
    return q_out, P_final


# speedup vs baseline: 1.5327x; 1.3878x over previous
import jax
import jax.numpy as jnp
from jax import lax
from jax.experimental import pallas as pl
from jax.experimental.pallas import tpu as pltpu

_B, _L, _D = 4, 1024, 256
_C = 256
_NC = _L // _C
_BB = 2

_PREC = None


def _bdot_tt(a, b):
    return lax.dot_general(a, b, (((2,), (2,)), ((0,), (0,))),
                           preferred_element_type=jnp.float32,
                           precision=_PREC)


def _body(q_ref, k_ref, pprev_ref, gain_ref, oscale_ref,
          qout_ref, pfin_ref, p_scr):
    c = pl.program_id(1)

    @pl.when(c == 0)
    def _():
        p_scr[...] = pprev_ref[...]

    P = p_scr[...]
    F_in = jnp.sum(P * P, axis=(1, 2), keepdims=True)
    Q = q_ref[...]
    K = k_ref[...]
    QK = jnp.concatenate([Q, K], axis=1)

    row = lax.broadcasted_iota(jnp.int32, (_C, _C), 0)
    col = lax.broadcasted_iota(jnp.int32, (_C, _C), 1)

    A = _bdot_tt(QK, P)
    num_inter = A[:, :_C]
    d = jnp.sum(A[:, _C:] * K, axis=2, keepdims=True)

    T = _bdot_tt(QK, K)
    S = T[:, :_C]
    G = T[:, _C:]
    S_causal = jnp.where(col <= row, S, 0.0)
    num_intra = lax.dot_general(S_causal, K, (((2,), (1,)), ((0,), (0,))),
                                preferred_element_type=jnp.float32,
                                precision=_PREC)
    numer = num_inter + num_intra

    G2 = G * G
    W = jnp.where(col < row, 2.0, jnp.where(col == row, 1.0, 0.0))
    w_row = jnp.sum(G2 * W, axis=2, keepdims=True)

    tri = jnp.where(col <= row, jnp.float32(1.0), jnp.float32(0.0))
    tri_b = jnp.broadcast_to(tri, (_BB, _C, _C))
    cs = lax.dot_general(tri_b, 2.0 * d + w_row,
                         (((2,), (1,)), ((0,), (0,))),
                         preferred_element_type=jnp.float32,
                         precision=_PREC)
    fro = jnp.sqrt(F_in + cs)

    q_aligned = numer * (1.0 / (fro + 1e-7))
    gain = jnp.exp(gain_ref[...])
    qout_ref[...] = jnp.tanh(q_aligned * gain) * oscale_ref[...]

    P_new = P + lax.dot_general(K, K, (((1,), (1,)), ((0,), (0,))),
                                preferred_element_type=jnp.float32,
                                precision=_PREC)
    p_scr[...] = P_new

    @pl.when(c == _NC - 1)
    def _():
        pfin_ref[...] = P_new


@jax.jit
def kernel(q, k, P_prev, log_gain, output_scale):
    gain2d = log_gain.reshape(1, _D)
    oscale2d = output_scale.reshape(1, _D)

    q_out, P_final = pl.pallas_call(
        _body,
        out_shape=(
            jax.ShapeDtypeStruct((_B, _L, _D), jnp.float32),
            jax.ShapeDtypeStruct((_B, _D, _D), jnp.float32),
        ),
        grid=(_B // _BB, _NC),
        in_specs=[
            pl.BlockSpec((_BB, _C, _D), lambda b, c: (b, c, 0)),
            pl.BlockSpec((_BB, _C, _D), lambda b, c: (b, c, 0)),
            pl.BlockSpec((_BB, _D, _D), lambda b, c: (b, 0, 0)),
            pl.BlockSpec((1, _D), lambda b, c: (0, 0)),
            pl.BlockSpec((1, _D), lambda b, c: (0, 0)),
        ],
        out_specs=(
            pl.BlockSpec((_BB, _C, _D), lambda b, c: (b, c, 0)),
            pl.BlockSpec((_BB, _D, _D), lambda b, c: (b, 0, 0)),
        ),
        scratch_shapes=[
            pltpu.VMEM((_BB, _D, _D), jnp.float32),
        ],
        compiler_params=pltpu.CompilerParams(
            dimension_semantics=("parallel", "arbitrary"),
        ),
        name="qkproj_chunked",
    )(q, k, P_prev, gain2d, oscale2d)
    return q_out, P_final


# speedup vs baseline: 1.8001x; 1.1745x over previous
import jax
import jax.numpy as jnp
from jax import lax
from jax.experimental import pallas as pl
from jax.experimental.pallas import tpu as pltpu

_B, _L, _D = 4, 1024, 256
_C = 256
_NC = _L // _C
_BB = 4

_PREC = None


def _bdot_tt(a, b):
    return lax.dot_general(a, b, (((2,), (2,)), ((0,), (0,))),
                           preferred_element_type=jnp.float32,
                           precision=_PREC)


def _body(q_ref, k_ref, pprev_ref, gain_ref, oscale_ref,
          qout_ref, pfin_ref, p_scr):
    c = pl.program_id(1)

    @pl.when(c == 0)
    def _():
        p_scr[...] = pprev_ref[...]

    P = p_scr[...]
    F_in = jnp.sum(P * P, axis=(1, 2), keepdims=True)
    Q = q_ref[...]
    K = k_ref[...]
    QK = jnp.concatenate([Q, K], axis=1)

    row = lax.broadcasted_iota(jnp.int32, (_C, _C), 0)
    col = lax.broadcasted_iota(jnp.int32, (_C, _C), 1)

    A = _bdot_tt(QK, P)
    num_inter = A[:, :_C]
    d = jnp.sum(A[:, _C:] * K, axis=2, keepdims=True)

    T = _bdot_tt(QK, K)
    S = T[:, :_C]
    G = T[:, _C:]
    S_causal = jnp.where(col <= row, S, 0.0)
    num_intra = lax.dot_general(S_causal, K, (((2,), (1,)), ((0,), (0,))),
                                preferred_element_type=jnp.float32,
                                precision=_PREC)
    numer = num_inter + num_intra

    G2 = G * G
    W = jnp.where(col < row, 2.0, jnp.where(col == row, 1.0, 0.0))
    w_row = jnp.sum(G2 * W, axis=2, keepdims=True)

    tri = jnp.where(col <= row, jnp.float32(1.0), jnp.float32(0.0))
    tri_b = jnp.broadcast_to(tri, (_BB, _C, _C))
    cs = lax.dot_general(tri_b, 2.0 * d + w_row,
                         (((2,), (1,)), ((0,), (0,))),
                         preferred_element_type=jnp.float32,
                         precision=_PREC)
    fro = jnp.sqrt(F_in + cs)

    q_aligned = numer * (1.0 / (fro + 1e-7))
    gain = jnp.exp(gain_ref[...])
    qout_ref[...] = jnp.tanh(q_aligned * gain) * oscale_ref[...]

    P_new = P + lax.dot_general(K, K, (((1,), (1,)), ((0,), (0,))),
                                preferred_element_type=jnp.float32,
                                precision=_PREC)
    p_scr[...] = P_new

    @pl.when(c == _NC - 1)
    def _():
        pfin_ref[...] = P_new


@jax.jit
def kernel(q, k, P_prev, log_gain, output_scale):
    gain2d = log_gain.reshape(1, _D)
    oscale2d = output_scale.reshape(1, _D)

    q_out, P_final = pl.pallas_call(
        _body,
        out_shape=(
            jax.ShapeDtypeStruct((_B, _L, _D), jnp.float32),
            jax.ShapeDtypeStruct((_B, _D, _D), jnp.float32),
        ),
        grid=(_B // _BB, _NC),
        in_specs=[
            pl.BlockSpec((_BB, _C, _D), lambda b, c: (b, c, 0)),
            pl.BlockSpec((_BB, _C, _D), lambda b, c: (b, c, 0)),
            pl.BlockSpec((_BB, _D, _D), lambda b, c: (b, 0, 0)),
            pl.BlockSpec((1, _D), lambda b, c: (0, 0)),
            pl.BlockSpec((1, _D), lambda b, c: (0, 0)),
        ],
        out_specs=(
            pl.BlockSpec((_BB, _C, _D), lambda b, c: (b, c, 0)),
            pl.BlockSpec((_BB, _D, _D), lambda b, c: (b, 0, 0)),
        ),
        scratch_shapes=[
            pltpu.VMEM((_BB, _D, _D), jnp.float32),
        ],
        compiler_params=pltpu.CompilerParams(
            dimension_semantics=("parallel", "arbitrary"),
        ),
        name="qkproj_chunked",
    )(q, k, P_prev, gain2d, oscale2d)
    return q_out, P_final


# speedup vs baseline: 1.8751x; 1.0417x over previous
import jax
import jax.numpy as jnp
from jax import lax
from jax.experimental import pallas as pl
from jax.experimental.pallas import tpu as pltpu

_B, _L, _D = 4, 1024, 256
_C = 256
_NC = _L // _C
_BB = 4

_PREC = None


def _bdot_tt(a, b):
    return lax.dot_general(a, b, (((2,), (2,)), ((0,), (0,))),
                           preferred_element_type=jnp.float32,
                           precision=_PREC)


def _body(q_ref, k_ref, pprev_ref, gain_ref, oscale_ref,
          qout_ref, pfin_ref):
    c = pl.program_id(1)

    @pl.when(c == 0)
    def _():
        pfin_ref[...] = pprev_ref[...]

    P = pfin_ref[...]
    F_in = jnp.sum(P * P, axis=(1, 2), keepdims=True)
    Q = q_ref[...]
    K = k_ref[...]

    row = lax.broadcasted_iota(jnp.int32, (_C, _C), 0)
    col = lax.broadcasted_iota(jnp.int32, (_C, _C), 1)

    num_inter = _bdot_tt(Q, P)
    KP = _bdot_tt(K, P)
    d = jnp.sum(KP * K, axis=2, keepdims=True)

    S = _bdot_tt(Q, K)
    G = _bdot_tt(K, K)
    S_causal = jnp.where(col <= row, S, 0.0)
    num_intra = lax.dot_general(S_causal, K, (((2,), (1,)), ((0,), (0,))),
                                preferred_element_type=jnp.float32,
                                precision=_PREC)
    numer = num_inter + num_intra

    G2 = G * G
    W = jnp.where(col < row, 2.0, jnp.where(col == row, 1.0, 0.0))
    w_row = jnp.sum(G2 * W, axis=2, keepdims=True)

    tri = jnp.where(col <= row, jnp.float32(1.0), jnp.float32(0.0))
    tri_b = jnp.broadcast_to(tri, (_BB, _C, _C))
    cs = lax.dot_general(tri_b, 2.0 * d + w_row,
                         (((2,), (1,)), ((0,), (0,))),
                         preferred_element_type=jnp.float32,
                         precision=_PREC)
    fro = jnp.sqrt(F_in + cs)

    q_aligned = numer * (1.0 / (fro + 1e-7))
    gain = jnp.exp(gain_ref[...])
    qout_ref[...] = jnp.tanh(q_aligned * gain) * oscale_ref[...]

    pfin_ref[...] = P + lax.dot_general(K, K, (((1,), (1,)), ((0,), (0,))),
                                        preferred_element_type=jnp.float32,
                                        precision=_PREC)


@jax.jit
def kernel(q, k, P_prev, log_gain, output_scale):
    gain2d = log_gain.reshape(1, _D)
    oscale2d = output_scale.reshape(1, _D)

    q_out, P_final = pl.pallas_call(
        _body,
        out_shape=(
            jax.ShapeDtypeStruct((_B, _L, _D), jnp.float32),
            jax.ShapeDtypeStruct((_B, _D, _D), jnp.float32),
        ),
        grid=(_B // _BB, _NC),
        in_specs=[
            pl.BlockSpec((_BB, _C, _D), lambda b, c: (b, c, 0)),
            pl.BlockSpec((_BB, _C, _D), lambda b, c: (b, c, 0)),
            pl.BlockSpec((_BB, _D, _D), lambda b, c: (b, 0, 0)),
            pl.BlockSpec((1, _D), lambda b, c: (0, 0)),
            pl.BlockSpec((1, _D), lambda b, c: (0, 0)),
        ],
        out_specs=(
            pl.BlockSpec((_BB, _C, _D), lambda b, c: (b, c, 0)),
            pl.BlockSpec((_BB, _D, _D), lambda b, c: (b, 0, 0)),
        ),
        compiler_params=pltpu.CompilerParams(
            dimension_semantics=("parallel", "arbitrary"),
        ),
        name="qkproj_chunked",
    )(q, k, P_prev, gain2d, oscale2d)
    return q_out, P_final
